# baseline (speedup 1.0000x reference)
import jax
import jax.numpy as jnp
from jax.experimental import pallas as pl
from jax.experimental.pallas import tpu as pltpu

B = 128
D = 1024
EMB = 1024
C = 1000
H = D // 2


def _sample(key, probs):
    logits = jnp.broadcast_to(jnp.log(probs), (EMB, probs.shape[-1]))
    return jax.random.categorical(key, logits, axis=-1)


def _toggle_masks():
    availabilities = jnp.ones((1, 6), dtype=jnp.float32)
    p1 = jnp.ones((1, 4), dtype=jnp.float32) / 4.0
    p2 = jnp.ones((1, 6), dtype=jnp.float32) / 6.0
    sel1 = p1 * availabilities[:, :-2]
    sel1 = sel1 / jnp.sum(sel1, axis=-1, keepdims=True)
    sel2 = p2 * availabilities
    sel2 = sel2 / jnp.sum(sel2, axis=-1, keepdims=True)
    k1, k2 = jax.random.split(jax.random.key(42))
    m1 = jax.nn.one_hot(_sample(k1, sel1), 4, dtype=jnp.float32).T
    m2 = jax.nn.one_hot(_sample(k2, sel2), 6, dtype=jnp.float32).T
    return m1, m2


def _dot(a, b):
    return jax.lax.dot_general(
        a.astype(jnp.bfloat16), b.astype(jnp.bfloat16),
        dimension_numbers=(((1,), (0,)), ((), ())),
        preferred_element_type=jnp.float32)


def _body(x1_hbm, x2_hbm, w1_hbm, w2_hbm, wll2_hbm,
          b1_ref, m1_ref, b2_ref, m2_ref, wa_ref, bll2_ref,
          out_hbm, out1_hbm, ws_hbm,
          x1v, x2v, w1v, w2v, wll2v, o1v, wsv_s, outv, sems):
    def cp(i, src, dst):
        c = pltpu.make_async_copy(src, dst, sems.at[i])
        c.start()
        return c

    cp_x1 = cp(0, x1_hbm, x1v)
    cp_x2 = cp(1, x2_hbm, x2v)
    w1cps = []
    for m in range(4):
        for h in range(2):
            sl = pl.ds(h * H, H)
            w1cps.append(cp(2 + 2 * m + h, w1_hbm.at[m, sl, :],
                            w1v.at[m, sl, :]))
    w2cps = []
    w2order = [4, 5, 0, 1, 2, 3]
    for i, j in enumerate(w2order):
        for h in range(2):
            sl = pl.ds(h * H, H)
            w2cps.append(cp(10 + 2 * i + h, w2_hbm.at[j, sl, :],
                            w2v.at[j, sl, :]))
    llcps = [cp(22 + h, wll2_hbm.at[pl.ds(h * H, H), :],
                wll2v.at[pl.ds(h * H, H), :]) for h in range(2)]

    cp_x1.wait()
    cp_x2.wait()
    wsv = jnp.sum(x2v[...] * wa_ref[...][:, :, None], axis=0)
    wsv_s[...] = wsv
    cp_ws = cp(24, wsv_s, ws_hbm)

    acc1 = jnp.zeros((B, EMB), jnp.float32)
    for m in range(4):
        for h in range(2):
            w1cps[2 * m + h].wait()
            acc1 += _dot(x1v[m, :, h * H:(h + 1) * H],
                         w1v[m, h * H:(h + 1) * H, :]) * m1_ref[m:m + 1, :]
    bg1 = jnp.sum(b1_ref[...] * m1_ref[...], axis=0, keepdims=True)
    o1 = jax.nn.relu(acc1 + bg1)
    o1v[...] = o1
    cp_o1 = cp(25, o1v, out1_hbm)

    acc2 = jnp.zeros((B, EMB), jnp.float32)
    for i, j in enumerate(w2order):
        for h in range(2):
            w2cps[2 * i + h].wait()
            if j < 4:
                src_h = x2v[j, :, h * H:(h + 1) * H]
            elif j == 4:
                src_h = o1[:, h * H:(h + 1) * H]
            else:
                src_h = wsv[:, h * H:(h + 1) * H]
            acc2 += _dot(src_h,
                         w2v[j, h * H:(h + 1) * H, :]) * m2_ref[j:j + 1, :]
    bg2 = jnp.sum(b2_ref[...] * m2_ref[...], axis=0, keepdims=True)
    hh = jax.nn.relu(acc2 + bg2)
    llcps[0].wait()
    outp = _dot(hh[:, 0:H], wll2v[0:H, :]) + bll2_ref[...]
    llcps[1].wait()
    outv[...] = outp + _dot(hh[:, H:D], wll2v[H:D, :])
    cp_out = cp(26, outv, out_hbm)

    cp_ws.wait()
    cp_o1.wait()
    cp_out.wait()


def kernel(outputs1, outputs2, available, W_dock1, b_dock1, W_dock2, b_dock2,
           ws_weights, W_ll2, b_ll2):
    del available
    m1, m2 = _toggle_masks()
    wa = (ws_weights / jnp.sum(ws_weights)).reshape(4, 1)

    out, out1, wsout = pl.pallas_call(
        _body,
        grid=(1,),
        in_specs=[
            pl.BlockSpec(memory_space=pl.ANY),
            pl.BlockSpec(memory_space=pl.ANY),
            pl.BlockSpec(memory_space=pl.ANY),
            pl.BlockSpec(memory_space=pl.ANY),
            pl.BlockSpec(memory_space=pl.ANY),
            pl.BlockSpec((4, EMB), lambda k: (0, 0)),
            pl.BlockSpec((4, EMB), lambda k: (0, 0)),
            pl.BlockSpec((6, EMB), lambda k: (0, 0)),
            pl.BlockSpec((6, EMB), lambda k: (0, 0)),
            pl.BlockSpec((4, 1), lambda k: (0, 0)),
            pl.BlockSpec((1, C), lambda k: (0, 0)),
        ],
        out_specs=[
            pl.BlockSpec(memory_space=pl.ANY),
            pl.BlockSpec(memory_space=pl.ANY),
            pl.BlockSpec(memory_space=pl.ANY),
        ],
        out_shape=[
            jax.ShapeDtypeStruct((B, C), jnp.float32),
            jax.ShapeDtypeStruct((B, EMB), jnp.float32),
            jax.ShapeDtypeStruct((B, EMB), jnp.float32),
        ],
        scratch_shapes=[
            pltpu.VMEM((4, B, D), jnp.float32),
            pltpu.VMEM((4, B, D), jnp.float32),
            pltpu.VMEM((4, D, EMB), jnp.float32),
            pltpu.VMEM((6, D, EMB), jnp.float32),
            pltpu.VMEM((D, C), jnp.float32),
            pltpu.VMEM((B, EMB), jnp.float32),
            pltpu.VMEM((B, EMB), jnp.float32),
            pltpu.VMEM((B, C), jnp.float32),
            pltpu.SemaphoreType.DMA((27,)),
        ],
        compiler_params=pltpu.CompilerParams(
            dimension_semantics=("arbitrary",),
            vmem_limit_bytes=100 * 1024 * 1024),
    )(outputs1, outputs2, W_dock1, W_dock2, W_ll2,
      b_dock1, m1, b_dock2, m2, wa, b_ll2.reshape(1, C))

    return (out, out1, wsout)


# speedup vs baseline: 1.0029x; 1.0029x over previous
import jax
import jax.numpy as jnp
from jax.experimental import pallas as pl
from jax.experimental.pallas import tpu as pltpu

B = 128
D = 1024
EMB = 1024
C = 1000


def _sample(key, probs):
    logits = jnp.broadcast_to(jnp.log(probs), (EMB, probs.shape[-1]))
    return jax.random.categorical(key, logits, axis=-1)


def _toggle_masks():
    availabilities = jnp.ones((1, 6), dtype=jnp.float32)
    p1 = jnp.ones((1, 4), dtype=jnp.float32) / 4.0
    p2 = jnp.ones((1, 6), dtype=jnp.float32) / 6.0
    sel1 = p1 * availabilities[:, :-2]
    sel1 = sel1 / jnp.sum(sel1, axis=-1, keepdims=True)
    sel2 = p2 * availabilities
    sel2 = sel2 / jnp.sum(sel2, axis=-1, keepdims=True)
    k1, k2 = jax.random.split(jax.random.key(42))
    m1 = jax.nn.one_hot(_sample(k1, sel1), 4, dtype=jnp.float32).T
    m2 = jax.nn.one_hot(_sample(k2, sel2), 6, dtype=jnp.float32).T
    return m1, m2


def _dot(a, b):
    return jax.lax.dot_general(
        a.astype(jnp.bfloat16), b.astype(jnp.bfloat16),
        dimension_numbers=(((1,), (0,)), ((), ())),
        preferred_element_type=jnp.float32)


def _body(x1_hbm, x2_hbm, w1_hbm, w2_hbm, wll2_hbm,
          b1_ref, m1_ref, b2_ref, m2_ref, wa_ref, bll2_ref,
          out_hbm, out1_hbm, ws_hbm,
          x1v, x2v, w1v, w2v, wll2v, o1v, wsv_s, outv, sems):
    cp_x1 = pltpu.make_async_copy(x1_hbm, x1v, sems.at[0])
    cp_x1.start()
    cp_x2 = pltpu.make_async_copy(x2_hbm, x2v, sems.at[1])
    cp_x2.start()
    cps = []
    for m in range(4):
        cp = pltpu.make_async_copy(w1_hbm.at[m], w1v.at[m], sems.at[2 + m])
        cp.start()
        cps.append(cp)
    w2cps = []
    for j in range(6):
        cp = pltpu.make_async_copy(w2_hbm.at[j], w2v.at[j], sems.at[6 + j])
        cp.start()
        w2cps.append(cp)
    cp_ll = pltpu.make_async_copy(wll2_hbm, wll2v, sems.at[12])
    cp_ll.start()

    cp_x1.wait()
    cp_x2.wait()
    wsv = jnp.sum(x2v[...] * wa_ref[...][:, :, None], axis=0)
    wsv_s[...] = wsv
    cp_ws = pltpu.make_async_copy(wsv_s, ws_hbm, sems.at[13])
    cp_ws.start()

    acc1 = jnp.zeros((B, EMB), jnp.float32)
    for m in range(4):
        cps[m].wait()
        acc1 += _dot(x1v[m], w1v[m]) * m1_ref[m:m + 1, :]
    bg1 = jnp.sum(b1_ref[...] * m1_ref[...], axis=0, keepdims=True)
    o1 = jax.nn.relu(acc1 + bg1)
    o1v[...] = o1
    cp_o1 = pltpu.make_async_copy(o1v, out1_hbm, sems.at[14])
    cp_o1.start()

    acc2 = jnp.zeros((B, EMB), jnp.float32)
    for j in range(4):
        w2cps[j].wait()
        acc2 += _dot(x2v[j], w2v[j]) * m2_ref[j:j + 1, :]
    w2cps[4].wait()
    acc2 += _dot(o1, w2v[4]) * m2_ref[4:5, :]
    w2cps[5].wait()
    acc2 += _dot(wsv, w2v[5]) * m2_ref[5:6, :]
    bg2 = jnp.sum(b2_ref[...] * m2_ref[...], axis=0, keepdims=True)
    h = jax.nn.relu(acc2 + bg2)
    cp_ll.wait()
    outv[...] = _dot(h, wll2v[...]) + bll2_ref[...]
    cp_out = pltpu.make_async_copy(outv, out_hbm, sems.at[15])
    cp_out.start()

    cp_ws.wait()
    cp_o1.wait()
    cp_out.wait()


def kernel(outputs1, outputs2, available, W_dock1, b_dock1, W_dock2, b_dock2,
           ws_weights, W_ll2, b_ll2):
    del available
    m1, m2 = _toggle_masks()
    wa = (ws_weights / jnp.sum(ws_weights)).reshape(4, 1)

    out, out1, wsout = pl.pallas_call(
        _body,
        grid=(1,),
        in_specs=[
            pl.BlockSpec(memory_space=pl.ANY),
            pl.BlockSpec(memory_space=pl.ANY),
            pl.BlockSpec(memory_space=pl.ANY),
            pl.BlockSpec(memory_space=pl.ANY),
            pl.BlockSpec(memory_space=pl.ANY),
            pl.BlockSpec((4, EMB), lambda k: (0, 0)),
            pl.BlockSpec((4, EMB), lambda k: (0, 0)),
            pl.BlockSpec((6, EMB), lambda k: (0, 0)),
            pl.BlockSpec((6, EMB), lambda k: (0, 0)),
            pl.BlockSpec((4, 1), lambda k: (0, 0)),
            pl.BlockSpec((1, C), lambda k: (0, 0)),
        ],
        out_specs=[
            pl.BlockSpec(memory_space=pl.ANY),
            pl.BlockSpec(memory_space=pl.ANY),
            pl.BlockSpec(memory_space=pl.ANY),
        ],
        out_shape=[
            jax.ShapeDtypeStruct((B, C), jnp.float32),
            jax.ShapeDtypeStruct((B, EMB), jnp.float32),
            jax.ShapeDtypeStruct((B, EMB), jnp.float32),
        ],
        scratch_shapes=[
            pltpu.VMEM((4, B, D), jnp.float32),
            pltpu.VMEM((4, B, D), jnp.float32),
            pltpu.VMEM((4, D, EMB), jnp.float32),
            pltpu.VMEM((6, D, EMB), jnp.float32),
            pltpu.VMEM((D, C), jnp.float32),
            pltpu.VMEM((B, EMB), jnp.float32),
            pltpu.VMEM((B, EMB), jnp.float32),
            pltpu.VMEM((B, C), jnp.float32),
            pltpu.SemaphoreType.DMA((16,)),
        ],
        compiler_params=pltpu.CompilerParams(
            dimension_semantics=("arbitrary",),
            vmem_limit_bytes=100 * 1024 * 1024),
    )(outputs1, outputs2, W_dock1, W_dock2, W_ll2,
      b_dock1, m1, b_dock2, m2, wa, b_ll2.reshape(1, C))

    return (out, out1, wsout)


# speedup vs baseline: 1.0254x; 1.0224x over previous
import jax
import jax.numpy as jnp
from jax.experimental import pallas as pl
from jax.experimental.pallas import tpu as pltpu

B = 128
D = 1024
EMB = 1024
C = 1000


def _sample(key, probs):
    logits = jnp.broadcast_to(jnp.log(probs), (EMB, probs.shape[-1]))
    return jax.random.categorical(key, logits, axis=-1)


def _toggle_masks():
    availabilities = jnp.ones((1, 6), dtype=jnp.float32)
    p1 = jnp.ones((1, 4), dtype=jnp.float32) / 4.0
    p2 = jnp.ones((1, 6), dtype=jnp.float32) / 6.0
    sel1 = p1 * availabilities[:, :-2]
    sel1 = sel1 / jnp.sum(sel1, axis=-1, keepdims=True)
    sel2 = p2 * availabilities
    sel2 = sel2 / jnp.sum(sel2, axis=-1, keepdims=True)
    k1, k2 = jax.random.split(jax.random.key(42))
    m1 = jax.nn.one_hot(_sample(k1, sel1), 4, dtype=jnp.float32).T
    m2 = jax.nn.one_hot(_sample(k2, sel2), 6, dtype=jnp.float32).T
    return m1, m2


def _dot(a, b):
    return jax.lax.dot_general(
        a.astype(jnp.bfloat16), b.astype(jnp.bfloat16),
        dimension_numbers=(((1,), (0,)), ((), ())),
        preferred_element_type=jnp.float32)


def _body(x1_hbm, x2_hbm, w1_hbm, w2_hbm, wll2_hbm,
          b1_ref, m1_ref, b2_ref, m2_ref, wa_ref, bll2_ref,
          out_ref, out1_ref, ws_ref,
          x1v, x2v, w1v, w2v, wll2v, sems):
    cps = []
    for m in range(4):
        cp = pltpu.make_async_copy(w1_hbm.at[m], w1v.at[m], sems.at[m])
        cp.start()
        cps.append(cp)
    cp_x1 = pltpu.make_async_copy(x1_hbm, x1v, sems.at[4])
    cp_x1.start()
    cp_x2 = pltpu.make_async_copy(x2_hbm, x2v, sems.at[5])
    cp_x2.start()
    w2cps = []
    for j in range(6):
        cp = pltpu.make_async_copy(w2_hbm.at[j], w2v.at[j], sems.at[6 + j])
        cp.start()
        w2cps.append(cp)
    cp_ll = pltpu.make_async_copy(wll2_hbm, wll2v, sems.at[12])
    cp_ll.start()

    cp_x1.wait()
    cp_x2.wait()
    wsv = jnp.sum(x2v[...] * wa_ref[...][:, :, None], axis=0)
    ws_ref[...] = wsv

    acc1 = jnp.zeros((B, EMB), jnp.float32)
    for m in range(4):
        cps[m].wait()
        acc1 += _dot(x1v[m], w1v[m]) * m1_ref[m:m + 1, :]
    bg1 = jnp.sum(b1_ref[...] * m1_ref[...], axis=0, keepdims=True)
    o1 = jax.nn.relu(acc1 + bg1)
    out1_ref[...] = o1

    acc2 = jnp.zeros((B, EMB), jnp.float32)
    for j in range(4):
        w2cps[j].wait()
        acc2 += _dot(x2v[j], w2v[j]) * m2_ref[j:j + 1, :]
    w2cps[4].wait()
    acc2 += _dot(o1, w2v[4]) * m2_ref[4:5, :]
    w2cps[5].wait()
    acc2 += _dot(wsv, w2v[5]) * m2_ref[5:6, :]
    bg2 = jnp.sum(b2_ref[...] * m2_ref[...], axis=0, keepdims=True)
    h = jax.nn.relu(acc2 + bg2)
    cp_ll.wait()
    out_ref[...] = _dot(h, wll2v[...]) + bll2_ref[...]


def kernel(outputs1, outputs2, available, W_dock1, b_dock1, W_dock2, b_dock2,
           ws_weights, W_ll2, b_ll2):
    del available
    m1, m2 = _toggle_masks()
    wa = (ws_weights / jnp.sum(ws_weights)).reshape(4, 1)

    out, out1, wsout = pl.pallas_call(
        _body,
        grid=(1,),
        in_specs=[
            pl.BlockSpec(memory_space=pl.ANY),
            pl.BlockSpec(memory_space=pl.ANY),
            pl.BlockSpec(memory_space=pl.ANY),
            pl.BlockSpec(memory_space=pl.ANY),
            pl.BlockSpec(memory_space=pl.ANY),
            pl.BlockSpec((4, EMB), lambda k: (0, 0)),
            pl.BlockSpec((4, EMB), lambda k: (0, 0)),
            pl.BlockSpec((6, EMB), lambda k: (0, 0)),
            pl.BlockSpec((6, EMB), lambda k: (0, 0)),
            pl.BlockSpec((4, 1), lambda k: (0, 0)),
            pl.BlockSpec((1, C), lambda k: (0, 0)),
        ],
        out_specs=[
            pl.BlockSpec((B, C), lambda k: (0, 0)),
            pl.BlockSpec((B, EMB), lambda k: (0, 0)),
            pl.BlockSpec((B, EMB), lambda k: (0, 0)),
        ],
        out_shape=[
            jax.ShapeDtypeStruct((B, C), jnp.float32),
            jax.ShapeDtypeStruct((B, EMB), jnp.float32),
            jax.ShapeDtypeStruct((B, EMB), jnp.float32),
        ],
        scratch_shapes=[
            pltpu.VMEM((4, B, D), jnp.float32),
            pltpu.VMEM((4, B, D), jnp.float32),
            pltpu.VMEM((4, D, EMB), jnp.float32),
            pltpu.VMEM((6, D, EMB), jnp.float32),
            pltpu.VMEM((D, C), jnp.float32),
            pltpu.SemaphoreType.DMA((13,)),
        ],
        compiler_params=pltpu.CompilerParams(
            dimension_semantics=("arbitrary",),
            vmem_limit_bytes=100 * 1024 * 1024),
    )(outputs1, outputs2, W_dock1, W_dock2, W_ll2,
      b_dock1, m1, b_dock2, m2, wa, b_ll2.reshape(1, C))

    return (out, out1, wsout)
